# baseline (speedup 1.0000x reference)
import functools

import jax
import jax.numpy as jnp
from jax import lax
from jax.experimental import pallas as pl
from jax.experimental.pallas import tpu as pltpu
from jax.experimental.pallas import tpu_sc as plsc

NC = 2
NS = 16
L = 16
NW = NC * NS

B = 4096
S = 50
E = 64
O = 128

SEQ_PER_W = B // NW
SEQ_PER_CHUNK = 2
CHUNK_IDX = SEQ_PER_CHUNK * S
CHUNKS = SEQ_PER_W // SEQ_PER_CHUNK
IDX_ROWS_PER_W = SEQ_PER_W // SEQ_PER_CHUNK


def _sc_pool(table, idx2d):
  mesh = plsc.VectorSubcoreMesh(
      core_axis_name="c", subcore_axis_name="s", num_cores=NC, num_subcores=NS)

  @functools.partial(
      pl.kernel,
      out_type=jax.ShapeDtypeStruct((B, E), jnp.float32),
      mesh=mesh,
      scratch_types=[
          pltpu.VMEM((IDX_ROWS_PER_W, CHUNK_IDX), jnp.int32),
          pltpu.VMEM((CHUNK_IDX, E), jnp.float32),
          pltpu.VMEM((SEQ_PER_W, E), jnp.float32),
          pltpu.SemaphoreType.DMA,
      ],
      compiler_params=pltpu.CompilerParams(use_tc_tiling_on_sc=False),
  )
  def k(table_hbm, idx_hbm, out_hbm, idx_v, buf, pooled_v, sem):
    wid = lax.axis_index("s") * NC + lax.axis_index("c")
    pltpu.sync_copy(idx_hbm.at[pl.ds(wid * IDX_ROWS_PER_W, IDX_ROWS_PER_W)],
                    idx_v)

    def chunk_body(j, carry):
      pltpu.async_copy(table_hbm.at[idx_v.at[j]], buf, sem).wait()
      for s in range(SEQ_PER_CHUNK):
        accs = [jnp.zeros((L,), jnp.float32) for _ in range(E // L)]
        for t in range(S):
          r = s * S + t
          for d in range(E // L):
            accs[d] = accs[d] + buf[r, pl.ds(d * L, L)]
        for d in range(E // L):
          pooled_v[j * SEQ_PER_CHUNK + s, pl.ds(d * L, L)] = accs[d]
      return carry

    lax.fori_loop(0, CHUNKS, chunk_body, 0)
    pltpu.sync_copy(pooled_v, out_hbm.at[pl.ds(wid * SEQ_PER_W, SEQ_PER_W)])

  return k(table, idx2d)


def _tc_proj_body(x_ref, w_ref, b_ref, o_ref):
  x = x_ref[...] * jnp.float32(1.0 / S)
  o_ref[...] = jnp.tanh(
      jnp.dot(x, w_ref[...], preferred_element_type=jnp.float32) + b_ref[...])


def _tc_proj(pooled, W, b):
  blk = 512
  return pl.pallas_call(
      _tc_proj_body,
      grid=(B // blk,),
      in_specs=[
          pl.BlockSpec((blk, E), lambda i: (i, 0)),
          pl.BlockSpec((E, O), lambda i: (0, 0)),
          pl.BlockSpec((1, O), lambda i: (0, 0)),
      ],
      out_specs=pl.BlockSpec((blk, O), lambda i: (i, 0)),
      out_shape=jax.ShapeDtypeStruct((B, O), jnp.float32),
  )(pooled, W, b.reshape(1, O))


@jax.jit
def kernel(token_ids, table, W, b):
  idx2d = token_ids.astype(jnp.int32).reshape(-1, CHUNK_IDX)
  pooled = _sc_pool(table, idx2d)
  return _tc_proj(pooled, W, b)


# speedup vs baseline: 1.0643x; 1.0643x over previous
import functools

import jax
import jax.numpy as jnp
from jax import lax
from jax.experimental import pallas as pl
from jax.experimental.pallas import tpu as pltpu
from jax.experimental.pallas import tpu_sc as plsc

NC = 2
NS = 16
L = 16
NW = NC * NS

B = 4096
S = 50
E = 64
O = 128

SEQ_PER_W = B // NW


def _sc_pool(table, tids_t):
  mesh = plsc.VectorSubcoreMesh(
      core_axis_name="c", subcore_axis_name="s", num_cores=NC, num_subcores=NS)

  @functools.partial(
      pl.kernel,
      out_type=jax.ShapeDtypeStruct((B, E), jnp.float32),
      mesh=mesh,
      scratch_types=[
          pltpu.VMEM((S, SEQ_PER_W), jnp.int32),
          pltpu.VMEM((SEQ_PER_W, E), jnp.float32),
          pltpu.VMEM((SEQ_PER_W, E), jnp.float32),
          pltpu.VMEM((SEQ_PER_W, E), jnp.float32),
          pltpu.SemaphoreType.DMA,
          pltpu.SemaphoreType.DMA,
      ],
      compiler_params=pltpu.CompilerParams(use_tc_tiling_on_sc=False),
  )
  def k(tab_hbm, tid_hbm, out_hbm, idx_v, buf0, buf1, pooled_v, sem0, sem1):
    wid = lax.axis_index("s") * NC + lax.axis_index("c")
    base = wid * SEQ_PER_W
    pltpu.sync_copy(tid_hbm.at[pl.ds(0, S), pl.ds(base, SEQ_PER_W)], idx_v)

    def zero_body(j, carry):
      z = jnp.zeros((L,), jnp.float32)
      for d in range(E // L):
        pooled_v[j, pl.ds(d * L, L)] = z
      return carry

    lax.fori_loop(0, SEQ_PER_W, zero_body, 0)

    bufs = [buf0, buf1]
    sems = [sem0, sem1]

    def start(t):
      pltpu.async_copy(tab_hbm.at[idx_v.at[t]], bufs[t % 2], sems[t % 2])

    def accum(t):
      buf = bufs[t % 2]

      def acc_body(j, carry):
        for d in range(E // L):
          plsc.addupdate(pooled_v.at[j, pl.ds(d * L, L)],
                         buf[j, pl.ds(d * L, L)])
        return carry

      lax.fori_loop(0, SEQ_PER_W, acc_body, 0)

    start(0)
    start(1)
    for t in range(S):
      pltpu.make_async_copy(tab_hbm.at[idx_v.at[t]], bufs[t % 2],
                            sems[t % 2]).wait()
      accum(t)
      if t + 2 < S:
        start(t + 2)

    pltpu.sync_copy(pooled_v, out_hbm.at[pl.ds(base, SEQ_PER_W)])

  return k(table, tids_t)


def _tc_proj_body(x_ref, w_ref, b_ref, o_ref):
  x = x_ref[...] * jnp.float32(1.0 / S)
  o_ref[...] = jnp.tanh(
      jnp.dot(x, w_ref[...], preferred_element_type=jnp.float32) + b_ref[...])


def _tc_proj(pooled, W, b):
  blk = 512
  return pl.pallas_call(
      _tc_proj_body,
      grid=(B // blk,),
      in_specs=[
          pl.BlockSpec((blk, E), lambda i: (i, 0)),
          pl.BlockSpec((E, O), lambda i: (0, 0)),
          pl.BlockSpec((1, O), lambda i: (0, 0)),
      ],
      out_specs=pl.BlockSpec((blk, O), lambda i: (i, 0)),
      out_shape=jax.ShapeDtypeStruct((B, O), jnp.float32),
  )(pooled, W, b.reshape(1, O))


@jax.jit
def kernel(token_ids, table, W, b):
  tids_t = token_ids.astype(jnp.int32).T
  pooled = _sc_pool(table, tids_t)
  return _tc_proj(pooled, W, b)
